# Initial kernel scaffold; baseline (speedup 1.0000x reference)
import jax
import jax.numpy as jnp
from jax.experimental import pallas as pl


def kernel(pooled, spx):
    raise NotImplementedError("write your pallas kernel here")


# speedup vs baseline: 760.7813x; 760.7813x over previous
import functools

import jax
import jax.numpy as jnp
from jax import lax
from jax.experimental import pallas as pl
from jax.experimental.pallas import tpu as pltpu
from jax.experimental.pallas import tpu_sc as plsc

_NC = 2
_NS = 16
_NW = _NC * _NS

_CHUNK = 128


def _suppix_unpool_sc(pooled_flat, spx_flat, B, C, K, HW):
    tiles_per_b = _NW // B
    pix_per_tile = HW // tiles_per_b
    n_chunks = pix_per_tile // _CHUNK

    mesh = plsc.VectorSubcoreMesh(core_axis_name="c", subcore_axis_name="s")

    @functools.partial(
        pl.kernel,
        mesh=mesh,
        out_type=jax.ShapeDtypeStruct((B, C, HW), jnp.float32),
        compiler_params=pltpu.CompilerParams(
            needs_layout_passes=False,
            use_tc_tiling_on_sc=False,
        ),
        scratch_types=[
            pltpu.VMEM((C, K), jnp.float32),
            pltpu.VMEM((_CHUNK,), jnp.int32),
            pltpu.VMEM((C, _CHUNK), jnp.float32),
        ],
    )
    def k(pooled_hbm, spx_hbm, out_hbm, table_v, idx_v, obuf_v):
        wid = lax.axis_index("s") * _NC + lax.axis_index("c")
        b = wid // tiles_per_b
        t = wid % tiles_per_b
        pix0 = t * pix_per_tile

        pltpu.sync_copy(pooled_hbm.at[b], table_v)

        def chunk_body(g, _):
            col = pix0 + g * _CHUNK
            pltpu.sync_copy(spx_hbm.at[b, pl.ds(col, _CHUNK)], idx_v)
            for i in range(_CHUNK // 16):
                iv = idx_v[pl.ds(i * 16, 16)]
                for c in range(C):
                    cv = jnp.full((16,), c, dtype=jnp.int32)
                    vals = plsc.load_gather(table_v, [cv, iv])
                    obuf_v[c, pl.ds(i * 16, 16)] = vals
            pltpu.sync_copy(obuf_v, out_hbm.at[b, :, pl.ds(col, _CHUNK)])
            return 0

        lax.fori_loop(0, n_chunks, chunk_body, 0)

    return k(pooled_flat, spx_flat)


def kernel(pooled, spx):
    B, C, K = pooled.shape
    _, H, W = spx.shape
    HW = H * W
    out = _suppix_unpool_sc(pooled, spx.reshape(B, HW), B, C, K, HW)
    return out.reshape(B, C, H, W)


# speedup vs baseline: 1313.4997x; 1.7265x over previous
import functools

import jax
import jax.numpy as jnp
from jax import lax
from jax.experimental import pallas as pl
from jax.experimental.pallas import tpu as pltpu
from jax.experimental.pallas import tpu_sc as plsc

_NC = 2
_NS = 16
_NW = _NC * _NS

_CHUNK = 128
_G = 8


def _suppix_unpool_sc(pooled, spx_flat, B, C, K, HW):
    tiles_per_b = _NW // B
    pix_per_tile = HW // tiles_per_b
    n_chunks = pix_per_tile // _CHUNK

    mesh = plsc.VectorSubcoreMesh(core_axis_name="c", subcore_axis_name="s")

    @functools.partial(
        pl.kernel,
        mesh=mesh,
        out_type=jax.ShapeDtypeStruct((B, C, HW), jnp.float32),
        compiler_params=pltpu.CompilerParams(
            needs_layout_passes=False,
            use_tc_tiling_on_sc=False,
        ),
        scratch_types=[
            pltpu.VMEM((C, K), jnp.float32),
            pltpu.VMEM((2, _CHUNK), jnp.int32),
            pltpu.VMEM((2, C, _CHUNK), jnp.float32),
            pltpu.SemaphoreType.DMA,
            pltpu.SemaphoreType.DMA,
            pltpu.SemaphoreType.DMA,
            pltpu.SemaphoreType.DMA,
        ],
    )
    def k(pooled_hbm, spx_hbm, out_hbm, table_v, idx_v, obuf_v,
          isem0, isem1, osem0, osem1):
        isems = (isem0, isem1)
        osems = (osem0, osem1)
        wid = lax.axis_index("s") * _NC + lax.axis_index("c")
        b = wid // tiles_per_b
        t = wid % tiles_per_b
        pix0 = t * pix_per_tile

        pltpu.sync_copy(pooled_hbm.at[b], table_v)
        pltpu.async_copy(
            spx_hbm.at[b, pl.ds(pix0, _CHUNK)], idx_v.at[0], isems[0]
        )

        def pair_body(p, _):
            for s in (0, 1):
                g = 2 * p + s
                col = pix0 + g * _CHUNK
                pltpu.make_async_copy(
                    spx_hbm.at[b, pl.ds(col, _CHUNK)], idx_v.at[s], isems[s]
                ).wait()

                @pl.when(g + 1 < n_chunks)
                def _prefetch():
                    ncol = col + _CHUNK
                    pltpu.async_copy(
                        spx_hbm.at[b, pl.ds(ncol, _CHUNK)],
                        idx_v.at[1 - s],
                        isems[1 - s],
                    )

                @pl.when(g >= 2)
                def _drain():
                    pcol = col - 2 * _CHUNK
                    pltpu.make_async_copy(
                        obuf_v.at[s],
                        out_hbm.at[b, :, pl.ds(pcol, _CHUNK)],
                        osems[s],
                    ).wait()

                for i in range(_CHUNK // 16):
                    iv = idx_v[s, pl.ds(i * 16, 16)]
                    for c0 in range(0, C, _G):
                        vals = [
                            plsc.load_gather(
                                table_v,
                                [jnp.full((16,), c0 + j, dtype=jnp.int32), iv],
                            )
                            for j in range(_G)
                        ]
                        for j in range(_G):
                            obuf_v[s, c0 + j, pl.ds(i * 16, 16)] = vals[j]

                pltpu.async_copy(
                    obuf_v.at[s], out_hbm.at[b, :, pl.ds(col, _CHUNK)], osems[s]
                )
            return 0

        lax.fori_loop(0, n_chunks // 2, pair_body, 0)

        for s in (0, 1):
            g = n_chunks - 2 + s
            col = pix0 + g * _CHUNK
            pltpu.make_async_copy(
                obuf_v.at[s], out_hbm.at[b, :, pl.ds(col, _CHUNK)], osems[s]
            ).wait()

    return k(pooled, spx_flat)


def kernel(pooled, spx):
    B, C, K = pooled.shape
    _, H, W = spx.shape
    HW = H * W
    out = _suppix_unpool_sc(pooled, spx.reshape(B, HW), B, C, K, HW)
    return out.reshape(B, C, H, W)
